# baseline (speedup 1.0000x reference)
import functools

import jax
import jax.numpy as jnp
from jax import lax
from jax.experimental import pallas as pl
from jax.experimental.pallas import tpu as pltpu
from jax.experimental.pallas import tpu_sc as plsc

ND = 8
BATCH = 4096
DIM = 512
EPS = 1e-3
NC = 2
NS = 16
RPW = BATCH // NS
CPW = DIM // NC
L = 16
CHUNKS = CPW // L
TAB = ND * CPW
SLICE = TAB // NS
CTAB = ND * L

_f32 = jnp.float32
_i32 = jnp.int32


def _rsqrt(v):
    i = plsc.bitcast(v, _i32)
    i = jnp.int32(0x5F3759DF) - (i >> 1)
    y = plsc.bitcast(i, _f32)
    for _ in range(3):
        y = y * (1.5 - 0.5 * v * y * y)
    return y


def _row_domain(di_v, r):
    return di_v[pl.ds(r, L)][0]


def _body(x_hbm, di_hbm, gg_hbm, gb_hbm, dg_hbm, db_hbm, out_hbm,
          x_v, di_v, sum_v, ss_v, cnt_v, buf_v,
          gg_v, gb_v, dg_v, db_v, scale_v, bias_v,
          dom_s, ord_s, cnt_s, off_s,
          sh_sum, sh_ss, sh_cnt, sh_csum, sh_css, sh_ccnt, sem, gsem):
    cid = lax.axis_index("c")
    sid = lax.axis_index("s")
    r0 = sid * RPW
    c0 = cid * CPW

    hx = pltpu.async_copy(x_hbm.at[pl.ds(r0, RPW), pl.ds(c0, CPW)], x_v, sem)
    hg = [pltpu.async_copy(gg_hbm.at[pl.ds(c0, CPW)], gg_v, gsem),
          pltpu.async_copy(gb_hbm.at[pl.ds(c0, CPW)], gb_v, gsem),
          pltpu.async_copy(dg_hbm.at[:, pl.ds(c0, CPW)], dg_v, gsem),
          pltpu.async_copy(db_hbm.at[:, pl.ds(c0, CPW)], db_v, gsem)]
    pltpu.sync_copy(di_hbm.at[pl.ds(r0, RPW)], di_v.at[pl.ds(0, RPW)])

    zeros = jnp.zeros((L,), _f32)

    for d in range(ND):
        cnt_s[d] = jnp.int32(0)

    def count_body(r, dom):
        dom_next = _row_domain(di_v, r + 1)
        dom_s[r] = dom
        cnt_s[dom] = cnt_s[dom] + 1
        return dom_next

    lax.fori_loop(0, RPW, count_body, _row_domain(di_v, 0))

    acc = jnp.int32(0)
    for d in range(ND):
        off_s[d] = acc
        acc = acc + cnt_s[d]

    def scatter_body(r, carry):
        dom = dom_s[r]
        p = off_s[dom]
        off_s[dom] = p + 1
        ord_s[p] = r
        return carry

    lax.fori_loop(0, RPW, scatter_body, 0)

    hx.wait()

    for d in range(ND):
        start = jnp.int32(0) if d == 0 else off_s[d - 1]
        end = off_s[d]

        def stat_inner(j, accs):
            rid = ord_s[j]
            out = []
            for c in range(CHUNKS):
                xv = x_v[rid, pl.ds(c * L, L)]
                out.append(accs[2 * c] + xv)
                out.append(accs[2 * c + 1] + xv * xv)
            return tuple(out)

        accs = lax.fori_loop(start, end, stat_inner, (zeros,) * (2 * CHUNKS))
        for c in range(CHUNKS):
            sum_v[pl.ds(d * CPW + c * L, L)] = accs[2 * c]
            ss_v[pl.ds(d * CPW + c * L, L)] = accs[2 * c + 1]
        cnt_v[pl.ds(d * L, L)] = jnp.full((L,), (end - start).astype(_f32))

    pltpu.sync_copy(sum_v, sh_sum.at[sid])
    pltpu.sync_copy(ss_v, sh_ss.at[sid])
    pltpu.sync_copy(cnt_v, sh_cnt.at[sid])

    plsc.subcore_barrier()

    def reduce_body(c, carry):
        a = buf_v[0, pl.ds(c * L, L)]
        for j in range(1, NS):
            a = a + buf_v[j, pl.ds(c * L, L)]
        sum_v[pl.ds(c * L, L)] = a
        return carry

    off = sid * SLICE
    pltpu.sync_copy(sh_sum.at[:, pl.ds(off, SLICE)], buf_v)
    lax.fori_loop(0, SLICE // L, reduce_body, 0)
    pltpu.sync_copy(sum_v.at[pl.ds(0, SLICE)], sh_csum.at[pl.ds(off, SLICE)])

    pltpu.sync_copy(sh_ss.at[:, pl.ds(off, SLICE)], buf_v)
    lax.fori_loop(0, SLICE // L, reduce_body, 0)
    pltpu.sync_copy(sum_v.at[pl.ds(0, SLICE)], sh_css.at[pl.ds(off, SLICE)])

    @pl.when(sid == 0)
    def _():
        pltpu.sync_copy(sh_cnt, buf_v.at[:, pl.ds(0, CTAB)])
        lax.fori_loop(0, CTAB // L, reduce_body, 0)
        pltpu.sync_copy(sum_v.at[pl.ds(0, CTAB)], sh_ccnt)

    plsc.subcore_barrier()

    pltpu.sync_copy(sh_csum, sum_v)
    pltpu.sync_copy(sh_css, ss_v)
    pltpu.sync_copy(sh_ccnt, cnt_v)
    for h in hg:
        h.wait()

    def table_body(i, carry):
        d = i // CHUNKS
        co = (i - d * CHUNKS) * L
        o = i * L
        cn = cnt_v[pl.ds(d * L, L)]
        rc = 1.0 / jnp.maximum(cn, 1.0)
        mean = sum_v[pl.ds(o, L)] * rc
        var = ss_v[pl.ds(o, L)] * rc - mean * mean + EPS
        y = _rsqrt(var)
        sc = (gg_v[pl.ds(co, L)] + dg_v[d, pl.ds(co, L)]) * y
        scale_v[pl.ds(o, L)] = sc
        bias_v[pl.ds(o, L)] = (gb_v[pl.ds(co, L)]
                               + db_v[d, pl.ds(co, L)] - mean * sc)
        return carry

    lax.fori_loop(0, ND * CHUNKS, table_body, 0)

    for d in range(ND):
        start = jnp.int32(0) if d == 0 else off_s[d - 1]
        end = off_s[d]
        svs = [scale_v[pl.ds(d * CPW + c * L, L)] for c in range(CHUNKS)]
        bvs = [bias_v[pl.ds(d * CPW + c * L, L)] for c in range(CHUNKS)]

        def apply_inner(j, carry, svs=svs, bvs=bvs):
            rid = ord_s[j]
            for c in range(CHUNKS):
                x_v[rid, pl.ds(c * L, L)] = (
                    x_v[rid, pl.ds(c * L, L)] * svs[c] + bvs[c])
            return carry

        lax.fori_loop(start, end, apply_inner, 0)

    pltpu.sync_copy(x_v, out_hbm.at[pl.ds(r0, RPW), pl.ds(c0, CPW)])


@functools.partial(pl.kernel,
                   out_type=jax.ShapeDtypeStruct((BATCH, DIM), _f32),
                   mesh=plsc.VectorSubcoreMesh(
                       core_axis_name="c", subcore_axis_name="s",
                       num_cores=NC, num_subcores=NS),
                   compiler_params=pltpu.CompilerParams(
                       needs_layout_passes=False,
                       skip_device_barrier=True),
                   scratch_types=[
                       pltpu.VMEM((RPW, CPW), _f32),
                       pltpu.VMEM((RPW + L,), _i32),
                       pltpu.VMEM((TAB,), _f32),
                       pltpu.VMEM((TAB,), _f32),
                       pltpu.VMEM((CTAB,), _f32),
                       pltpu.VMEM((NS, SLICE), _f32),
                       pltpu.VMEM((CPW,), _f32),
                       pltpu.VMEM((CPW,), _f32),
                       pltpu.VMEM((ND, CPW), _f32),
                       pltpu.VMEM((ND, CPW), _f32),
                       pltpu.VMEM((TAB,), _f32),
                       pltpu.VMEM((TAB,), _f32),
                       pltpu.SMEM((RPW,), _i32),
                       pltpu.SMEM((RPW,), _i32),
                       pltpu.SMEM((ND,), _i32),
                       pltpu.SMEM((ND,), _i32),
                       pltpu.VMEM_SHARED((NS, TAB), _f32),
                       pltpu.VMEM_SHARED((NS, TAB), _f32),
                       pltpu.VMEM_SHARED((NS, CTAB), _f32),
                       pltpu.VMEM_SHARED((TAB,), _f32),
                       pltpu.VMEM_SHARED((TAB,), _f32),
                       pltpu.VMEM_SHARED((CTAB,), _f32),
                       pltpu.SemaphoreType.DMA,
                       pltpu.SemaphoreType.DMA,
                   ])
def _pn_call(x_hbm, di_hbm, gg_hbm, gb_hbm, dg_hbm, db_hbm, out_hbm,
             *scratch):
    _body(x_hbm, di_hbm, gg_hbm, gb_hbm, dg_hbm, db_hbm, out_hbm, *scratch)


@jax.jit
def kernel(inputs, global_gamma, global_beta, domain_gamma, domain_beta,
           domain_index):
    di = domain_index.astype(_i32)
    return _pn_call(inputs, di, global_gamma, global_beta,
                    domain_gamma, domain_beta)


# speedup vs baseline: 1.0749x; 1.0749x over previous
import functools

import jax
import jax.numpy as jnp
from jax import lax
from jax.experimental import pallas as pl
from jax.experimental.pallas import tpu as pltpu
from jax.experimental.pallas import tpu_sc as plsc

ND = 8
BATCH = 4096
DIM = 512
EPS = 1e-3
NC = 2
NS = 16
RPW = BATCH // NS
CPW = DIM // NC
L = 16
CHUNKS = CPW // L
TAB = ND * CPW
SLICE = TAB // NS
CTAB = ND * L

_f32 = jnp.float32
_i32 = jnp.int32


def _rsqrt(v):
    i = plsc.bitcast(v, _i32)
    i = jnp.int32(0x5F3759DF) - (i >> 1)
    y = plsc.bitcast(i, _f32)
    for _ in range(3):
        y = y * (1.5 - 0.5 * v * y * y)
    return y


def _row_domain(di_v, r):
    return di_v[pl.ds(r, L)][0]


def _body(x_hbm, di_hbm, gg_hbm, gb_hbm, dg_hbm, db_hbm, out_hbm,
          x_v, di_v, sum_v, ss_v, cnt_v, buf_v,
          gg_v, gb_v, dg_v, db_v, scale_v, bias_v,
          dom_s, ord_s, cnt_s, off_s,
          sh_sum, sh_ss, sh_cnt, sh_csum, sh_css, sh_ccnt, sem, gsem, osem):
    cid = lax.axis_index("c")
    sid = lax.axis_index("s")
    r0 = sid * RPW
    c0 = cid * CPW

    hx = pltpu.async_copy(x_hbm.at[pl.ds(r0, RPW), pl.ds(c0, CPW)], x_v, sem)
    hg = [pltpu.async_copy(gg_hbm.at[pl.ds(c0, CPW)], gg_v, gsem),
          pltpu.async_copy(gb_hbm.at[pl.ds(c0, CPW)], gb_v, gsem),
          pltpu.async_copy(dg_hbm.at[:, pl.ds(c0, CPW)], dg_v, gsem),
          pltpu.async_copy(db_hbm.at[:, pl.ds(c0, CPW)], db_v, gsem)]
    pltpu.sync_copy(di_hbm.at[pl.ds(r0, RPW)], di_v.at[pl.ds(0, RPW)])

    zeros = jnp.zeros((L,), _f32)

    for d in range(ND):
        cnt_s[d] = jnp.int32(0)

    def count_body(r, dom):
        dom_next = _row_domain(di_v, r + 1)
        dom_s[r] = dom
        cnt_s[dom] = cnt_s[dom] + 1
        return dom_next

    lax.fori_loop(0, RPW, count_body, _row_domain(di_v, 0))

    acc = jnp.int32(0)
    for d in range(ND):
        off_s[d] = acc
        acc = acc + cnt_s[d]

    def scatter_body(r, carry):
        dom = dom_s[r]
        p = off_s[dom]
        off_s[dom] = p + 1
        ord_s[p] = r
        return carry

    lax.fori_loop(0, RPW, scatter_body, 0)

    hx.wait()

    for d in range(ND):
        start = jnp.int32(0) if d == 0 else off_s[d - 1]
        end = off_s[d]

        def stat_inner(j, accs):
            rid = ord_s[j]
            out = []
            for c in range(CHUNKS):
                xv = x_v[rid, pl.ds(c * L, L)]
                out.append(accs[2 * c] + xv)
                out.append(accs[2 * c + 1] + xv * xv)
            return tuple(out)

        accs = lax.fori_loop(start, end, stat_inner, (zeros,) * (2 * CHUNKS))
        for c in range(CHUNKS):
            sum_v[pl.ds(d * CPW + c * L, L)] = accs[2 * c]
            ss_v[pl.ds(d * CPW + c * L, L)] = accs[2 * c + 1]
        cnt_v[pl.ds(d * L, L)] = jnp.full((L,), (end - start).astype(_f32))

    pltpu.sync_copy(sum_v, sh_sum.at[sid])
    pltpu.sync_copy(ss_v, sh_ss.at[sid])
    pltpu.sync_copy(cnt_v, sh_cnt.at[sid])

    plsc.subcore_barrier()

    def reduce_body(c, carry):
        a = buf_v[0, pl.ds(c * L, L)]
        for j in range(1, NS):
            a = a + buf_v[j, pl.ds(c * L, L)]
        sum_v[pl.ds(c * L, L)] = a
        return carry

    off = sid * SLICE
    pltpu.sync_copy(sh_sum.at[:, pl.ds(off, SLICE)], buf_v)
    lax.fori_loop(0, SLICE // L, reduce_body, 0)
    pltpu.sync_copy(sum_v.at[pl.ds(0, SLICE)], sh_csum.at[pl.ds(off, SLICE)])

    pltpu.sync_copy(sh_ss.at[:, pl.ds(off, SLICE)], buf_v)
    lax.fori_loop(0, SLICE // L, reduce_body, 0)
    pltpu.sync_copy(sum_v.at[pl.ds(0, SLICE)], sh_css.at[pl.ds(off, SLICE)])

    @pl.when(sid == 0)
    def _():
        pltpu.sync_copy(sh_cnt, buf_v.at[:, pl.ds(0, CTAB)])
        lax.fori_loop(0, CTAB // L, reduce_body, 0)
        pltpu.sync_copy(sum_v.at[pl.ds(0, CTAB)], sh_ccnt)

    plsc.subcore_barrier()

    pltpu.sync_copy(sh_csum, sum_v)
    pltpu.sync_copy(sh_css, ss_v)
    pltpu.sync_copy(sh_ccnt, cnt_v)
    for h in hg:
        h.wait()

    def table_body(i, carry):
        d = i // CHUNKS
        co = (i - d * CHUNKS) * L
        o = i * L
        cn = cnt_v[pl.ds(d * L, L)]
        rc = 1.0 / jnp.maximum(cn, 1.0)
        mean = sum_v[pl.ds(o, L)] * rc
        var = ss_v[pl.ds(o, L)] * rc - mean * mean + EPS
        y = _rsqrt(var)
        sc = (gg_v[pl.ds(co, L)] + dg_v[d, pl.ds(co, L)]) * y
        scale_v[pl.ds(o, L)] = sc
        bias_v[pl.ds(o, L)] = (gb_v[pl.ds(co, L)]
                               + db_v[d, pl.ds(co, L)] - mean * sc)
        return carry

    tb0 = sid * (ND * CHUNKS // NS)
    lax.fori_loop(tb0, tb0 + ND * CHUNKS // NS, table_body, 0)
    pltpu.sync_copy(scale_v.at[pl.ds(tb0 * L, SLICE)],
                    sh_csum.at[pl.ds(tb0 * L, SLICE)])
    pltpu.sync_copy(bias_v.at[pl.ds(tb0 * L, SLICE)],
                    sh_css.at[pl.ds(tb0 * L, SLICE)])
    plsc.subcore_barrier()
    pltpu.sync_copy(sh_csum, scale_v)
    pltpu.sync_copy(sh_css, bias_v)

    for d in range(ND):
        start = jnp.int32(0) if d == 0 else off_s[d - 1]
        end = off_s[d]
        svs = [scale_v[pl.ds(d * CPW + c * L, L)] for c in range(CHUNKS)]
        bvs = [bias_v[pl.ds(d * CPW + c * L, L)] for c in range(CHUNKS)]

        def apply_inner(j, carry, svs=svs, bvs=bvs):
            rid = ord_s[j]
            for c in range(CHUNKS):
                x_v[rid, pl.ds(c * L, L)] = (
                    x_v[rid, pl.ds(c * L, L)] * svs[c] + bvs[c])
            pltpu.async_copy(x_v.at[rid],
                             out_hbm.at[r0 + rid, pl.ds(c0, CPW)], osem)
            return carry

        lax.fori_loop(start, end, apply_inner, 0)

    pltpu.make_async_copy(
        x_hbm.at[pl.ds(r0, RPW), pl.ds(c0, CPW)], x_v, osem).wait()


@functools.partial(pl.kernel,
                   out_type=jax.ShapeDtypeStruct((BATCH, DIM), _f32),
                   mesh=plsc.VectorSubcoreMesh(
                       core_axis_name="c", subcore_axis_name="s",
                       num_cores=NC, num_subcores=NS),
                   compiler_params=pltpu.CompilerParams(
                       needs_layout_passes=False),
                   scratch_types=[
                       pltpu.VMEM((RPW, CPW), _f32),
                       pltpu.VMEM((RPW + L,), _i32),
                       pltpu.VMEM((TAB,), _f32),
                       pltpu.VMEM((TAB,), _f32),
                       pltpu.VMEM((CTAB,), _f32),
                       pltpu.VMEM((NS, SLICE), _f32),
                       pltpu.VMEM((CPW,), _f32),
                       pltpu.VMEM((CPW,), _f32),
                       pltpu.VMEM((ND, CPW), _f32),
                       pltpu.VMEM((ND, CPW), _f32),
                       pltpu.VMEM((TAB,), _f32),
                       pltpu.VMEM((TAB,), _f32),
                       pltpu.SMEM((RPW,), _i32),
                       pltpu.SMEM((RPW,), _i32),
                       pltpu.SMEM((ND,), _i32),
                       pltpu.SMEM((ND,), _i32),
                       pltpu.VMEM_SHARED((NS, TAB), _f32),
                       pltpu.VMEM_SHARED((NS, TAB), _f32),
                       pltpu.VMEM_SHARED((NS, CTAB), _f32),
                       pltpu.VMEM_SHARED((TAB,), _f32),
                       pltpu.VMEM_SHARED((TAB,), _f32),
                       pltpu.VMEM_SHARED((CTAB,), _f32),
                       pltpu.SemaphoreType.DMA,
                       pltpu.SemaphoreType.DMA,
                       pltpu.SemaphoreType.DMA,
                   ])
def _pn_call(x_hbm, di_hbm, gg_hbm, gb_hbm, dg_hbm, db_hbm, out_hbm,
             *scratch):
    _body(x_hbm, di_hbm, gg_hbm, gb_hbm, dg_hbm, db_hbm, out_hbm, *scratch)


@jax.jit
def kernel(inputs, global_gamma, global_beta, domain_gamma, domain_beta,
           domain_index):
    di = domain_index.astype(_i32)
    return _pn_call(inputs, di, global_gamma, global_beta,
                    domain_gamma, domain_beta)


# speedup vs baseline: 1.0804x; 1.0052x over previous
import functools

import jax
import jax.numpy as jnp
from jax import lax
from jax.experimental import pallas as pl
from jax.experimental.pallas import tpu as pltpu
from jax.experimental.pallas import tpu_sc as plsc

ND = 8
BATCH = 4096
DIM = 512
EPS = 1e-3
NC = 2
NS = 16
RPW = BATCH // NS
CPW = DIM // NC
L = 16
CHUNKS = CPW // L
TAB = ND * CPW
SLICE = TAB // NS
CTAB = ND * L

_f32 = jnp.float32
_i32 = jnp.int32


def _rsqrt(v):
    i = plsc.bitcast(v, _i32)
    i = jnp.int32(0x5F3759DF) - (i >> 1)
    y = plsc.bitcast(i, _f32)
    for _ in range(3):
        y = y * (1.5 - 0.5 * v * y * y)
    return y


def _row_domain(di_v, r):
    return di_v[pl.ds(r, L)][0]


def _body(x_hbm, di_hbm, gam_hbm, out_hbm,
          x_v, di_v, sum_v, ss_v, cnt_v, buf_v,
          gam_v, scale_v, bias_v,
          dom_s, ord_s, cnt_s, off_s,
          sh_sum, sh_ss, sh_cnt, sh_csum, sh_css, sh_ccnt, sem, gsem, osem):
    cid = lax.axis_index("c")
    sid = lax.axis_index("s")
    r0 = sid * RPW
    c0 = cid * CPW

    hx = pltpu.async_copy(x_hbm.at[pl.ds(r0, RPW), pl.ds(c0, CPW)], x_v, sem)
    hg = pltpu.async_copy(gam_hbm.at[:, :, pl.ds(c0, CPW)], gam_v, gsem)
    pltpu.sync_copy(di_hbm.at[pl.ds(r0, RPW)], di_v.at[pl.ds(0, RPW)])

    zeros = jnp.zeros((L,), _f32)

    for d in range(ND):
        cnt_s[d] = jnp.int32(0)

    def count_body(r, dom):
        dom_next = _row_domain(di_v, r + 1)
        dom_s[r] = dom
        cnt_s[dom] = cnt_s[dom] + 1
        return dom_next

    lax.fori_loop(0, RPW, count_body, _row_domain(di_v, 0))

    acc = jnp.int32(0)
    for d in range(ND):
        off_s[d] = acc
        acc = acc + cnt_s[d]

    def scatter_body(r, carry):
        dom = dom_s[r]
        p = off_s[dom]
        off_s[dom] = p + 1
        ord_s[p] = r
        return carry

    lax.fori_loop(0, RPW, scatter_body, 0)

    hx.wait()

    for d in range(ND):
        start = jnp.int32(0) if d == 0 else off_s[d - 1]
        end = off_s[d]

        def stat_inner(j, accs):
            rid = ord_s[j]
            out = []
            for c in range(CHUNKS):
                xv = x_v[rid, pl.ds(c * L, L)]
                out.append(accs[2 * c] + xv)
                out.append(accs[2 * c + 1] + xv * xv)
            return tuple(out)

        accs = lax.fori_loop(start, end, stat_inner, (zeros,) * (2 * CHUNKS))
        for c in range(CHUNKS):
            sum_v[pl.ds(d * CPW + c * L, L)] = accs[2 * c]
            ss_v[pl.ds(d * CPW + c * L, L)] = accs[2 * c + 1]
        cnt_v[pl.ds(d * L, L)] = jnp.full((L,), (end - start).astype(_f32))

    pltpu.sync_copy(sum_v, sh_sum.at[sid])
    pltpu.sync_copy(ss_v, sh_ss.at[sid])
    pltpu.sync_copy(cnt_v, sh_cnt.at[sid])

    plsc.subcore_barrier()

    def reduce_body(c, carry):
        a = buf_v[0, pl.ds(c * L, L)]
        for j in range(1, NS):
            a = a + buf_v[j, pl.ds(c * L, L)]
        sum_v[pl.ds(c * L, L)] = a
        return carry

    off = sid * SLICE
    pltpu.sync_copy(sh_sum.at[:, pl.ds(off, SLICE)], buf_v)
    lax.fori_loop(0, SLICE // L, reduce_body, 0)
    pltpu.sync_copy(sum_v.at[pl.ds(0, SLICE)], sh_csum.at[pl.ds(off, SLICE)])

    pltpu.sync_copy(sh_ss.at[:, pl.ds(off, SLICE)], buf_v)
    lax.fori_loop(0, SLICE // L, reduce_body, 0)
    pltpu.sync_copy(sum_v.at[pl.ds(0, SLICE)], sh_css.at[pl.ds(off, SLICE)])

    @pl.when(sid == 0)
    def _():
        pltpu.sync_copy(sh_cnt, buf_v.at[:, pl.ds(0, CTAB)])
        lax.fori_loop(0, CTAB // L, reduce_body, 0)
        pltpu.sync_copy(sum_v.at[pl.ds(0, CTAB)], sh_ccnt)

    plsc.subcore_barrier()

    pltpu.sync_copy(sh_csum, sum_v)
    pltpu.sync_copy(sh_css, ss_v)
    pltpu.sync_copy(sh_ccnt, cnt_v)
    hg.wait()

    def table_body(i, carry):
        d = i // CHUNKS
        co = (i - d * CHUNKS) * L
        o = i * L
        cn = cnt_v[pl.ds(d * L, L)]
        rc = 1.0 / jnp.maximum(cn, 1.0)
        mean = sum_v[pl.ds(o, L)] * rc
        var = ss_v[pl.ds(o, L)] * rc - mean * mean + EPS
        y = _rsqrt(var)
        sc = (gam_v[0, d, pl.ds(co, L)] + gam_v[2, d, pl.ds(co, L)]) * y
        scale_v[pl.ds(o, L)] = sc
        bias_v[pl.ds(o, L)] = (gam_v[1, d, pl.ds(co, L)]
                               + gam_v[3, d, pl.ds(co, L)] - mean * sc)
        return carry

    tb0 = sid * (ND * CHUNKS // NS)
    lax.fori_loop(tb0, tb0 + ND * CHUNKS // NS, table_body, 0)
    pltpu.sync_copy(scale_v.at[pl.ds(tb0 * L, SLICE)],
                    sh_csum.at[pl.ds(tb0 * L, SLICE)])
    pltpu.sync_copy(bias_v.at[pl.ds(tb0 * L, SLICE)],
                    sh_css.at[pl.ds(tb0 * L, SLICE)])
    plsc.subcore_barrier()
    pltpu.sync_copy(sh_csum, scale_v)
    pltpu.sync_copy(sh_css, bias_v)

    for d in range(ND):
        start = jnp.int32(0) if d == 0 else off_s[d - 1]
        end = off_s[d]
        svs = [scale_v[pl.ds(d * CPW + c * L, L)] for c in range(CHUNKS)]
        bvs = [bias_v[pl.ds(d * CPW + c * L, L)] for c in range(CHUNKS)]

        def apply_inner(j, carry, svs=svs, bvs=bvs):
            rid = ord_s[j]
            for c in range(CHUNKS):
                x_v[rid, pl.ds(c * L, L)] = (
                    x_v[rid, pl.ds(c * L, L)] * svs[c] + bvs[c])
            pltpu.async_copy(x_v.at[rid],
                             out_hbm.at[r0 + rid, pl.ds(c0, CPW)], osem)
            return carry

        lax.fori_loop(start, end, apply_inner, 0)

    pltpu.make_async_copy(
        x_hbm.at[pl.ds(r0, RPW), pl.ds(c0, CPW)], x_v, osem).wait()


@functools.partial(pl.kernel,
                   out_type=jax.ShapeDtypeStruct((BATCH, DIM), _f32),
                   mesh=plsc.VectorSubcoreMesh(
                       core_axis_name="c", subcore_axis_name="s",
                       num_cores=NC, num_subcores=NS),
                   compiler_params=pltpu.CompilerParams(
                       needs_layout_passes=False),
                   scratch_types=[
                       pltpu.VMEM((RPW, CPW), _f32),
                       pltpu.VMEM((RPW + L,), _i32),
                       pltpu.VMEM((TAB,), _f32),
                       pltpu.VMEM((TAB,), _f32),
                       pltpu.VMEM((CTAB,), _f32),
                       pltpu.VMEM((NS, SLICE), _f32),
                       pltpu.VMEM((4, ND, CPW), _f32),
                       pltpu.VMEM((TAB,), _f32),
                       pltpu.VMEM((TAB,), _f32),
                       pltpu.SMEM((RPW,), _i32),
                       pltpu.SMEM((RPW,), _i32),
                       pltpu.SMEM((ND,), _i32),
                       pltpu.SMEM((ND,), _i32),
                       pltpu.VMEM_SHARED((NS, TAB), _f32),
                       pltpu.VMEM_SHARED((NS, TAB), _f32),
                       pltpu.VMEM_SHARED((NS, CTAB), _f32),
                       pltpu.VMEM_SHARED((TAB,), _f32),
                       pltpu.VMEM_SHARED((TAB,), _f32),
                       pltpu.VMEM_SHARED((CTAB,), _f32),
                       pltpu.SemaphoreType.DMA,
                       pltpu.SemaphoreType.DMA,
                       pltpu.SemaphoreType.DMA,
                   ])
def _pn_call(x_hbm, di_hbm, gam_hbm, out_hbm, *scratch):
    _body(x_hbm, di_hbm, gam_hbm, out_hbm, *scratch)


@jax.jit
def kernel(inputs, global_gamma, global_beta, domain_gamma, domain_beta,
           domain_index):
    di = domain_index.astype(_i32)
    gam = jnp.stack([
        jnp.broadcast_to(global_gamma, (ND, DIM)),
        jnp.broadcast_to(global_beta, (ND, DIM)),
        domain_gamma, domain_beta])
    return _pn_call(inputs, di, gam)


# speedup vs baseline: 1.0968x; 1.0151x over previous
import functools

import jax
import jax.numpy as jnp
from jax import lax
from jax.experimental import pallas as pl
from jax.experimental.pallas import tpu as pltpu
from jax.experimental.pallas import tpu_sc as plsc

ND = 8
BATCH = 4096
DIM = 512
EPS = 1e-3
NC = 2
NS = 16
RPW = BATCH // NS
CPW = DIM // NC
L = 16
CHUNKS = CPW // L
TAB = ND * CPW
SLICE = TAB // NS
CTAB = ND * L

_f32 = jnp.float32
_i32 = jnp.int32


def _rsqrt(v):
    i = plsc.bitcast(v, _i32)
    i = jnp.int32(0x5F3759DF) - (i >> 1)
    y = plsc.bitcast(i, _f32)
    for _ in range(3):
        y = y * (1.5 - 0.5 * v * y * y)
    return y


def _row_domain(di_v, r):
    return di_v[pl.ds(r, L)][0]


def _body(x_hbm, di_hbm, gam_hbm, out_hbm,
          x_v, di_v, sum_v, ss_v, cnt_v, buf_v,
          gam_v, scale_v, bias_v,
          dom_s, ord_s, cnt_s, off_s,
          sh_sum, sh_ss, sh_cnt, sh_csum, sh_css, sem, gsem, osem):
    cid = lax.axis_index("c")
    sid = lax.axis_index("s")
    r0 = sid * RPW
    c0 = cid * CPW

    hx = pltpu.async_copy(x_hbm.at[pl.ds(r0, RPW), pl.ds(c0, CPW)], x_v, sem)
    hg = pltpu.async_copy(gam_hbm.at[:, :, pl.ds(c0, CPW)], gam_v, gsem)
    pltpu.sync_copy(di_hbm.at[pl.ds(r0, RPW)], di_v.at[pl.ds(0, RPW)])

    zeros = jnp.zeros((L,), _f32)

    for d in range(ND):
        cnt_s[d] = jnp.int32(0)

    def count_body(r, dom):
        dom_next = _row_domain(di_v, r + 1)
        dom_s[r] = dom
        cnt_s[dom] = cnt_s[dom] + 1
        return dom_next

    lax.fori_loop(0, RPW, count_body, _row_domain(di_v, 0))

    acc = jnp.int32(0)
    for d in range(ND):
        off_s[d] = acc
        acc = acc + cnt_s[d]

    def scatter_body(r, carry):
        dom = dom_s[r]
        p = off_s[dom]
        off_s[dom] = p + 1
        ord_s[p] = r
        return carry

    lax.fori_loop(0, RPW, scatter_body, 0)

    hx.wait()

    for d in range(ND):
        start = jnp.int32(0) if d == 0 else off_s[d - 1]
        end = off_s[d]

        def stat_inner(j, accs):
            rid = ord_s[j]
            out = []
            for c in range(CHUNKS):
                xv = x_v[rid, pl.ds(c * L, L)]
                out.append(accs[2 * c] + xv)
                out.append(accs[2 * c + 1] + xv * xv)
            return tuple(out)

        accs = lax.fori_loop(start, end, stat_inner, (zeros,) * (2 * CHUNKS))
        for c in range(CHUNKS):
            sum_v[pl.ds(d * CPW + c * L, L)] = accs[2 * c]
            ss_v[pl.ds(d * CPW + c * L, L)] = accs[2 * c + 1]
        cnt_v[pl.ds(d * L, L)] = jnp.full((L,), (end - start).astype(_f32))

    pltpu.sync_copy(sum_v, sh_sum.at[sid])
    pltpu.sync_copy(ss_v, sh_ss.at[sid])
    pltpu.sync_copy(cnt_v, sh_cnt.at[sid])

    plsc.subcore_barrier()

    def make_reduce(dst):
        def red(c, carry):
            a = buf_v[0, pl.ds(c * L, L)]
            for j in range(1, NS):
                a = a + buf_v[j, pl.ds(c * L, L)]
            dst[pl.ds(c * L, L)] = a
            return carry
        return red

    off = sid * SLICE
    pltpu.sync_copy(sh_sum.at[:, pl.ds(off, SLICE)], buf_v)
    lax.fori_loop(0, SLICE // L, make_reduce(sum_v), 0)
    pltpu.sync_copy(sh_ss.at[:, pl.ds(off, SLICE)], buf_v)
    lax.fori_loop(0, SLICE // L, make_reduce(ss_v), 0)
    pltpu.sync_copy(sh_cnt, buf_v.at[:, pl.ds(0, CTAB)])
    lax.fori_loop(0, CTAB // L, make_reduce(cnt_v), 0)
    hg.wait()

    def table_body(k, carry):
        i = sid * (SLICE // L) + k
        d = i // CHUNKS
        co = (i - d * CHUNKS) * L
        o = k * L
        cn = cnt_v[pl.ds(d * L, L)]
        rc = 1.0 / jnp.maximum(cn, 1.0)
        mean = sum_v[pl.ds(o, L)] * rc
        var = ss_v[pl.ds(o, L)] * rc - mean * mean + EPS
        y = _rsqrt(var)
        sc = (gam_v[0, d, pl.ds(co, L)] + gam_v[2, d, pl.ds(co, L)]) * y
        scale_v[pl.ds(o, L)] = sc
        bias_v[pl.ds(o, L)] = (gam_v[1, d, pl.ds(co, L)]
                               + gam_v[3, d, pl.ds(co, L)] - mean * sc)
        return carry

    lax.fori_loop(0, SLICE // L, table_body, 0)
    pltpu.sync_copy(scale_v.at[pl.ds(0, SLICE)], sh_csum.at[pl.ds(off, SLICE)])
    pltpu.sync_copy(bias_v.at[pl.ds(0, SLICE)], sh_css.at[pl.ds(off, SLICE)])
    plsc.subcore_barrier()
    pltpu.sync_copy(sh_csum, scale_v)
    pltpu.sync_copy(sh_css, bias_v)

    for d in range(ND):
        start = jnp.int32(0) if d == 0 else off_s[d - 1]
        end = off_s[d]
        svs = [scale_v[pl.ds(d * CPW + c * L, L)] for c in range(CHUNKS)]
        bvs = [bias_v[pl.ds(d * CPW + c * L, L)] for c in range(CHUNKS)]

        def apply_inner(j, carry, svs=svs, bvs=bvs):
            rid = ord_s[j]
            for c in range(CHUNKS):
                x_v[rid, pl.ds(c * L, L)] = (
                    x_v[rid, pl.ds(c * L, L)] * svs[c] + bvs[c])
            pltpu.async_copy(x_v.at[rid],
                             out_hbm.at[r0 + rid, pl.ds(c0, CPW)], osem)
            return carry

        lax.fori_loop(start, end, apply_inner, 0)

    pltpu.make_async_copy(
        x_hbm.at[pl.ds(r0, RPW), pl.ds(c0, CPW)], x_v, osem).wait()


@functools.partial(pl.kernel,
                   out_type=jax.ShapeDtypeStruct((BATCH, DIM), _f32),
                   mesh=plsc.VectorSubcoreMesh(
                       core_axis_name="c", subcore_axis_name="s",
                       num_cores=NC, num_subcores=NS),
                   compiler_params=pltpu.CompilerParams(
                       needs_layout_passes=False),
                   scratch_types=[
                       pltpu.VMEM((RPW, CPW), _f32),
                       pltpu.VMEM((RPW + L,), _i32),
                       pltpu.VMEM((TAB,), _f32),
                       pltpu.VMEM((TAB,), _f32),
                       pltpu.VMEM((CTAB,), _f32),
                       pltpu.VMEM((NS, SLICE), _f32),
                       pltpu.VMEM((4, ND, CPW), _f32),
                       pltpu.VMEM((TAB,), _f32),
                       pltpu.VMEM((TAB,), _f32),
                       pltpu.SMEM((RPW,), _i32),
                       pltpu.SMEM((RPW,), _i32),
                       pltpu.SMEM((ND,), _i32),
                       pltpu.SMEM((ND,), _i32),
                       pltpu.VMEM_SHARED((NS, TAB), _f32),
                       pltpu.VMEM_SHARED((NS, TAB), _f32),
                       pltpu.VMEM_SHARED((NS, CTAB), _f32),
                       pltpu.VMEM_SHARED((TAB,), _f32),
                       pltpu.VMEM_SHARED((TAB,), _f32),
                       pltpu.SemaphoreType.DMA,
                       pltpu.SemaphoreType.DMA,
                       pltpu.SemaphoreType.DMA,
                   ])
def _pn_call(x_hbm, di_hbm, gam_hbm, out_hbm, *scratch):
    _body(x_hbm, di_hbm, gam_hbm, out_hbm, *scratch)


@jax.jit
def kernel(inputs, global_gamma, global_beta, domain_gamma, domain_beta,
           domain_index):
    di = domain_index.astype(_i32)
    gam = jnp.stack([
        jnp.broadcast_to(global_gamma, (ND, DIM)),
        jnp.broadcast_to(global_beta, (ND, DIM)),
        domain_gamma, domain_beta])
    return _pn_call(inputs, di, gam)
